# Initial kernel scaffold; baseline (speedup 1.0000x reference)
import jax
import jax.numpy as jnp
from jax.experimental import pallas as pl


def kernel(x, W1, b1, g1, bt1, W2, b2, g2, bt2, gdec, bdec, edge_index):
    raise NotImplementedError("write your pallas kernel here")


# speedup vs baseline: 1.6493x; 1.6493x over previous
import jax
import jax.numpy as jnp
from jax import lax
from jax.experimental import pallas as pl
from jax.experimental.pallas import tpu as pltpu
from jax.experimental.pallas import tpu_sc as plsc

N = 10000
E = 320000
D_IN = 128
H = 256
Z = 64
EPS = 1e-5
NH = N // 2

NC = 2
NS = 16

EPAD = 327680
ERWS = EPAD // 128
TROWS = ERWS // NS
CH = 8
HCH = 4
NCH = TROWS // CH
A1_ROWS = NH + 8
A2_ROWS = N + 16

import functools


@functools.lru_cache(maxsize=1)
def _sc_mesh():
    return plsc.VectorSubcoreMesh(core_axis_name="c", subcore_axis_name="s",
                                  num_cores=NC, num_subcores=NS)


def _zero_acc(zeros_hbm, acc, s, acc_rows):
    zr = (acc_rows // NS) // 8 * 8
    pltpu.sync_copy(zeros_hbm.at[pl.ds(0, zr)], acc.at[pl.ds(s * zr, zr)])

    @pl.when(s == 0)
    def _tail():
        pltpu.sync_copy(zeros_hbm.at[pl.ds(0, acc_rows - NS * zr)],
                        acc.at[pl.ds(NS * zr, acc_rows - NS * zr)])


def _write_acc(acc, out, s, row0, nrows):
    wr = (nrows // NS) // 8 * 8
    pltpu.sync_copy(acc.at[pl.ds(s * wr, wr)],
                    out.at[pl.ds(row0 + s * wr, wr)])

    @pl.when(s == 0)
    def _tail():
        pltpu.sync_copy(acc.at[pl.ds(NS * wr, nrows - NS * wr)],
                        out.at[pl.ds(row0 + NS * wr, nrows - NS * wr)])


def _edge_sweep(src1d, dst1d, table_hbm, idx_s, idx_d, rows, sem, acc,
                src_off0, dst_off0, s, degacc=None, ones_v=None):
    def chunk(g, carry):
        e0 = (s * TROWS + g) * 128
        pltpu.sync_copy(src1d.at[pl.ds(src_off0 + e0, 128)], idx_s)
        pltpu.sync_copy(dst1d.at[pl.ds(dst_off0 + e0, 128)], idx_d)
        pltpu.async_copy(table_hbm.at[idx_s], rows, sem).wait()
        pltpu.sync_copy(rows, acc.at[idx_d], add=True)
        return carry

    lax.fori_loop(0, TROWS, chunk, 0)


def _deg_sweep(dst1d, idx_d, ones_v, acc, dst_off0, s):
    def chunk(g, carry):
        e0 = (s * TROWS + g) * 128
        pltpu.sync_copy(dst1d.at[pl.ds(dst_off0 + e0, 128)], idx_d)
        pltpu.sync_copy(ones_v, acc.at[idx_d], add=True)
        return carry

    lax.fori_loop(0, TROWS, chunk, 0)


def _agg1_body(src1d, dstn1d, x_hbm, zeros_hbm, ones_hbm,
               sums_out, deg_out,
               idx_s, idx_d, rows, ones_v, sem, acc):
    c = lax.axis_index("c")
    s = lax.axis_index("s")
    _zero_acc(zeros_hbm, acc, s, A1_ROWS)
    pltpu.sync_copy(ones_hbm, ones_v)
    plsc.subcore_barrier()
    _edge_sweep(src1d, dstn1d, x_hbm, idx_s, idx_d, rows, sem, acc,
                0, c * EPAD, s)
    plsc.subcore_barrier()
    _write_acc(acc, sums_out, s, c * NH, NH)
    plsc.subcore_barrier()
    _zero_acc(zeros_hbm, acc, s, A1_ROWS)
    plsc.subcore_barrier()
    _deg_sweep(dstn1d, idx_d, ones_v, acc, c * EPAD, s)
    plsc.subcore_barrier()
    _write_acc(acc, deg_out, s, c * NH, NH)


@functools.lru_cache(maxsize=1)
def _agg1():
    return pl.kernel(
        _agg1_body,
        out_type=(jax.ShapeDtypeStruct((N, D_IN), jnp.float32),
                  jax.ShapeDtypeStruct((N, 128), jnp.float32)),
        mesh=_sc_mesh(),
        scratch_types=[
            pltpu.VMEM((128,), jnp.int32),
            pltpu.VMEM((128,), jnp.int32),
            pltpu.VMEM((128, D_IN), jnp.float32),
            pltpu.VMEM((128, 128), jnp.float32),
            pltpu.SemaphoreType.DMA,
            pltpu.VMEM_SHARED((A1_ROWS, D_IN), jnp.float32),
        ],
    )


def _agg2_body(src1d2, dstn1d, h2_hbm, zeros_hbm, sums_out,
               idx_s, idx_d, rows, sem, acc):
    c = lax.axis_index("c")
    s = lax.axis_index("s")
    for p in range(2):
        _zero_acc(zeros_hbm, acc, s, A1_ROWS)
        plsc.subcore_barrier()
        _edge_sweep(src1d2, dstn1d, h2_hbm, idx_s, idx_d, rows, sem, acc,
                    c * EPAD, p * EPAD, s)
        plsc.subcore_barrier()
        _write_acc(acc, sums_out, s, c * N + p * NH, NH)
        plsc.subcore_barrier()


@functools.lru_cache(maxsize=1)
def _agg2():
    return pl.kernel(
        _agg2_body,
        out_type=jax.ShapeDtypeStruct((2 * N, 128), jnp.float32),
        mesh=_sc_mesh(),
        scratch_types=[
            pltpu.VMEM((128,), jnp.int32),
            pltpu.VMEM((128,), jnp.int32),
            pltpu.VMEM((128, 128), jnp.float32),
            pltpu.SemaphoreType.DMA,
            pltpu.VMEM_SHARED((A1_ROWS, 128), jnp.float32),
        ],
    )


def _enc1_body(sums_ref, deg_ref, w_ref, b_ref, g_ref, bt_ref,
               enc_ref, split_ref):
    rdeg = 1.0 / jnp.maximum(deg_ref[:, 0:1], 1.0)
    t = lax.dot_general(sums_ref[...] * rdeg, w_ref[...],
                        (((1,), (0,)), ((), ())),
                        preferred_element_type=jnp.float32)
    t = jnp.tanh(t + b_ref[...])
    mu = jnp.mean(t, axis=0, keepdims=True)
    var = jnp.mean(jnp.square(t - mu), axis=0, keepdims=True)
    e = (t - mu) * lax.rsqrt(var + EPS) * g_ref[...] + bt_ref[...]
    enc_ref[...] = e
    split_ref[0:N, :] = e[:, 0:128]
    split_ref[N:2 * N, :] = e[:, 128:256]


def _enc2_body(sums_ref, deg_ref, w_ref, b_ref, g_ref, bt_ref, enc_ref):
    rdeg = 1.0 / jnp.maximum(deg_ref[:, 0:1], 1.0)
    t = lax.dot_general(sums_ref[0:N, :] * rdeg, w_ref[0:128, :],
                        (((1,), (0,)), ((), ())),
                        preferred_element_type=jnp.float32)
    t += lax.dot_general(sums_ref[N:2 * N, :] * rdeg, w_ref[128:256, :],
                         (((1,), (0,)), ((), ())),
                         preferred_element_type=jnp.float32)
    t = jnp.tanh(t + b_ref[...])
    mu = jnp.mean(t, axis=0, keepdims=True)
    var = jnp.mean(jnp.square(t - mu), axis=0, keepdims=True)
    enc_ref[...] = (t - mu) * lax.rsqrt(var + EPS) * g_ref[...] + bt_ref[...]


BI = 400


def _dstats_body(zb_ref, zt_ref, stat_ref):
    i = pl.program_id(0)

    @pl.when(i == 0)
    def _init():
        stat_ref[...] = jnp.zeros_like(stat_ref)

    logits = lax.dot_general(zb_ref[...], zt_ref[...],
                             (((1,), (0,)), ((), ())),
                             preferred_element_type=jnp.float32)
    sg = jax.nn.sigmoid(logits)
    stat_ref[0:1, :] += jnp.sum(sg, axis=0, keepdims=True)
    stat_ref[1:2, :] += jnp.sum(sg * sg, axis=0, keepdims=True)


def _dnorm_body(zb_ref, zt_ref, stat_ref, g_ref, b_ref, out_ref):
    mu = stat_ref[0:1, :] * (1.0 / N)
    ex2 = stat_ref[1:2, :] * (1.0 / N)
    scale = lax.rsqrt(ex2 - mu * mu + EPS) * g_ref[...]
    logits = lax.dot_general(zb_ref[...], zt_ref[...],
                             (((1,), (0,)), ((), ())),
                             preferred_element_type=jnp.float32)
    sg = jax.nn.sigmoid(logits)
    out_ref[...] = (sg - mu) * scale + b_ref[...]


def _decoder(z, g2d, b2d):
    k = z.shape[1]
    zt = z.T
    stats = pl.pallas_call(
        _dstats_body,
        grid=(N // BI,),
        in_specs=[pl.BlockSpec((BI, k), lambda i: (i, 0)),
                  pl.BlockSpec((k, N), lambda i: (0, 0))],
        out_specs=pl.BlockSpec((8, N), lambda i: (0, 0)),
        out_shape=jax.ShapeDtypeStruct((8, N), jnp.float32),
    )(z, zt)
    return pl.pallas_call(
        _dnorm_body,
        grid=(N // BI,),
        in_specs=[pl.BlockSpec((BI, k), lambda i: (i, 0)),
                  pl.BlockSpec((k, N), lambda i: (0, 0)),
                  pl.BlockSpec((8, N), lambda i: (0, 0)),
                  pl.BlockSpec((1, N), lambda i: (0, 0)),
                  pl.BlockSpec((1, N), lambda i: (0, 0))],
        out_specs=pl.BlockSpec((BI, N), lambda i: (i, 0)),
        out_shape=jax.ShapeDtypeStruct((N, N), jnp.float32),
    )(z, zt, stats, g2d, b2d)


def kernel(x, W1, b1, g1, bt1, W2, b2, g2, bt2, gdec, bdec, edge_index):
    src = edge_index[0]
    dst = edge_index[1]
    pad = EPAD - E
    src_p = jnp.concatenate([src, jnp.zeros((pad,), jnp.int32)])
    dst_p = jnp.concatenate([dst, jnp.full((pad,), N, jnp.int32)])
    d0 = jnp.where(dst_p < NH, dst_p, NH)
    d1c = dst_p - NH
    d1 = jnp.where((d1c >= 0) & (d1c < NH), d1c, NH)
    dstn1d = jnp.concatenate([d0, d1])
    src1d2 = jnp.concatenate([src_p, src_p + N])
    zeros128 = jnp.zeros((624, 128), jnp.float32)
    ones128 = jnp.ones((128, 128), jnp.float32)

    sums1, deg = _agg1()(src_p, dstn1d, x, zeros128, ones128)

    enc1, enc1_split = pl.pallas_call(
        _enc1_body,
        out_shape=(jax.ShapeDtypeStruct((N, H), jnp.float32),
                   jax.ShapeDtypeStruct((2 * N, 128), jnp.float32)),
    )(sums1, deg, W1, b1.reshape(1, H), g1.reshape(1, H), bt1.reshape(1, H))

    sums2 = _agg2()(src1d2, dstn1d, enc1_split, zeros128)

    enc2 = pl.pallas_call(
        _enc2_body,
        out_shape=jax.ShapeDtypeStruct((N, Z), jnp.float32),
    )(sums2, deg, W2, b2.reshape(1, Z), g2.reshape(1, Z), bt2.reshape(1, Z))

    g2d = gdec.reshape(1, N)
    b2d = bdec.reshape(1, N)
    dec1 = _decoder(enc1, g2d, b2d)
    dec2 = _decoder(enc2, g2d, b2d)
    return enc2, dec1, dec2
